# baseline (speedup 1.0000x reference)
import jax
import jax.numpy as jnp
from jax.experimental import pallas as pl
from jax.experimental.pallas import tpu as pltpu

_B, _N, _C = 32, 8732, 21
_BB = 2
_STEPS = _B // _BB
_EPS = 1e-7


def _body(lam_ref, x_ref, y_ref, z_ref, sum_ref, cnt_ref):
    b = pl.program_id(0)

    @pl.when(b == 0)
    def _init():
        sum_ref[0, 0] = 0.0
        cnt_ref[0, 0] = 0.0

    lam = lam_ref[0]
    x = x_ref[...].astype(jnp.float32)
    y = y_ref[...].astype(jnp.float32)
    z = z_ref[...].astype(jnp.float32)

    mixed = lam * x + (1.0 - lam) * y + _EPS
    interp = z + _EPS
    p = (interp - mixed) * jnp.log(interp / mixed)

    lmax = jnp.max(x[:, 1:, :], axis=1, keepdims=True)
    rmax = jnp.max(y[:, 1:, :], axis=1, keepdims=True)
    mf = ((lmax > x[:, :1, :]) & (rmax > y[:, :1, :])).astype(jnp.float32)

    colsum = jnp.sum(p, axis=1, keepdims=True)
    sum_ref[0, 0] += jnp.sum(colsum * mf)
    cnt_ref[0, 0] += jnp.sum(mf)

    @pl.when(b == _STEPS - 1)
    def _fin():
        s = sum_ref[0, 0]
        c = cnt_ref[0, 0]
        sum_ref[0, 0] = jnp.where(c > 0.0, s / (2.0 * jnp.maximum(c, 1.0)), 0.0)


def kernel(lam, conf, conf_flip, loc, loc_flip, conf_shuffle,
           conf_interpolation, loc_shuffle, loc_interpolation):
    lam_arr = jnp.asarray(lam, jnp.float32).reshape(1)
    xt = jnp.transpose(conf.astype(jnp.bfloat16), (0, 2, 1))
    yt = jnp.transpose(conf_shuffle.astype(jnp.bfloat16), (0, 2, 1))
    zt = jnp.transpose(conf_interpolation.astype(jnp.bfloat16), (0, 2, 1))
    half_blocks = (_B // 2) // _BB
    out, _ = pl.pallas_call(
        _body,
        grid=(_STEPS,),
        in_specs=[
            pl.BlockSpec(memory_space=pltpu.SMEM),
            pl.BlockSpec((_BB, _C, _N), lambda b: (b, 0, 0)),
            pl.BlockSpec((_BB, _C, _N),
                         lambda b: (jax.lax.rem(b + half_blocks, _STEPS), 0, 0)),
            pl.BlockSpec((_BB, _C, _N), lambda b: (b, 0, 0)),
        ],
        out_specs=[
            pl.BlockSpec(memory_space=pltpu.SMEM),
            pl.BlockSpec(memory_space=pltpu.SMEM),
        ],
        out_shape=[
            jax.ShapeDtypeStruct((1, 1), jnp.float32),
            jax.ShapeDtypeStruct((1, 1), jnp.float32),
        ],
        compiler_params=pltpu.CompilerParams(
            dimension_semantics=("arbitrary",),
        ),
    )(lam_arr, xt, yt, zt)
    return out[0, 0]


# speedup vs baseline: 1.0341x; 1.0341x over previous
import jax
import jax.numpy as jnp
from jax.experimental import pallas as pl
from jax.experimental.pallas import tpu as pltpu

_B, _N, _C = 32, 8732, 21
_BB = 2
_STEPS = _B // _BB
_EPS = 1e-7


def _body(lam_ref, x_ref, y_ref, z_ref, sum_ref, cnt_ref):
    b = pl.program_id(0)

    @pl.when(b == 0)
    def _init():
        sum_ref[0, 0] = 0.0
        cnt_ref[0, 0] = 0.0

    lam_f = lam_ref[0]
    lam = lam_f.astype(jnp.bfloat16)
    om = (1.0 - lam_f).astype(jnp.bfloat16)
    x = x_ref[...]
    y = y_ref[...]
    z = z_ref[...]

    eps = jnp.bfloat16(_EPS)
    mixed = lam * x + om * y + eps
    interp = z + eps
    p = (interp - mixed).astype(jnp.float32) * jnp.log(
        (interp / mixed).astype(jnp.float32))

    lmax = jnp.max(x[:, 1:, :], axis=1, keepdims=True)
    rmax = jnp.max(y[:, 1:, :], axis=1, keepdims=True)
    mf = ((lmax > x[:, :1, :]) & (rmax > y[:, :1, :])).astype(jnp.float32)

    colsum = jnp.sum(p, axis=1, keepdims=True)
    sum_ref[0, 0] += jnp.sum(colsum * mf)
    cnt_ref[0, 0] += jnp.sum(mf)

    @pl.when(b == _STEPS - 1)
    def _fin():
        s = sum_ref[0, 0]
        c = cnt_ref[0, 0]
        sum_ref[0, 0] = jnp.where(c > 0.0, s / (2.0 * jnp.maximum(c, 1.0)), 0.0)


def kernel(lam, conf, conf_flip, loc, loc_flip, conf_shuffle,
           conf_interpolation, loc_shuffle, loc_interpolation):
    lam_arr = jnp.asarray(lam, jnp.float32).reshape(1)
    xt = jnp.transpose(conf.astype(jnp.bfloat16), (0, 2, 1))
    yt = jnp.transpose(conf_shuffle.astype(jnp.bfloat16), (0, 2, 1))
    zt = jnp.transpose(conf_interpolation.astype(jnp.bfloat16), (0, 2, 1))
    half_blocks = (_B // 2) // _BB
    out, _ = pl.pallas_call(
        _body,
        grid=(_STEPS,),
        in_specs=[
            pl.BlockSpec(memory_space=pltpu.SMEM),
            pl.BlockSpec((_BB, _C, _N), lambda b: (b, 0, 0)),
            pl.BlockSpec((_BB, _C, _N),
                         lambda b: (jax.lax.rem(b + half_blocks, _STEPS), 0, 0)),
            pl.BlockSpec((_BB, _C, _N), lambda b: (b, 0, 0)),
        ],
        out_specs=[
            pl.BlockSpec(memory_space=pltpu.SMEM),
            pl.BlockSpec(memory_space=pltpu.SMEM),
        ],
        out_shape=[
            jax.ShapeDtypeStruct((1, 1), jnp.float32),
            jax.ShapeDtypeStruct((1, 1), jnp.float32),
        ],
        compiler_params=pltpu.CompilerParams(
            dimension_semantics=("arbitrary",),
        ),
    )(lam_arr, xt, yt, zt)
    return out[0, 0]
